# Initial kernel scaffold; baseline (speedup 1.0000x reference)
import jax
import jax.numpy as jnp
from jax.experimental import pallas as pl


def kernel(x, edge_index, Wl1, Wr1, b1, Wl2, Wr2, b2, Wl3, Wr3, b3):
    raise NotImplementedError("write your pallas kernel here")


# speedup vs baseline: 4.4699x; 4.4699x over previous
import functools

import jax
import jax.numpy as jnp
from jax import lax
from jax.experimental import pallas as pl
from jax.experimental.pallas import tpu as pltpu
from jax.experimental.pallas import tpu_sc as plsc

N = 10000
D = 128
C16 = 16
NC = 2
NS = 16
NW = NC * NS
CHUNK = 128
CPT = 80
G = 8
EP = NW * CPT * CHUNK
NACC = N + 8
ZCH = 24
BLK = 1000


def _mesh():
    return plsc.VectorSubcoreMesh(
        core_axis_name="c", subcore_axis_name="s",
        num_cores=NC, num_subcores=NS)


def _fill(ref, rows, width, value):
    v = jnp.full((16,), value, jnp.float32)
    for i in range(rows):
        for j in range(width // 16):
            ref[i, pl.ds(j * 16, 16)] = v


def _zero_acc(acc, zbuf, s, width):
    for j in range(416 // NS):
        idx = s * (416 // NS) + j
        pltpu.sync_copy(zbuf, acc.at[pl.ds(idx * ZCH, ZCH)])

    @pl.when(s == NS - 1)
    def _():
        pltpu.sync_copy(zbuf, acc.at[pl.ds(416 * ZCH, ZCH)])


def _copy_out(acc, out_hbm, c, s, width):
    for j in range(416 // NS):
        idx = s * (416 // NS) + j
        pltpu.sync_copy(acc.at[pl.ds(idx * ZCH, ZCH)],
                        out_hbm.at[c, pl.ds(idx * ZCH, ZCH)])

    @pl.when(s == NS - 1)
    def _():
        pltpu.sync_copy(acc.at[pl.ds(416 * ZCH, 16)],
                        out_hbm.at[c, pl.ds(416 * ZCH, 16)])


def _make_agg(width):
    @functools.partial(
        pl.kernel,
        out_type=jax.ShapeDtypeStruct((NC, N, width), jnp.float32),
        mesh=_mesh(),
        compiler_params=pltpu.CompilerParams(
            use_tc_tiling_on_sc=(width == D)),
        scratch_types=[
            pltpu.VMEM((G, CHUNK), jnp.int32),
            pltpu.VMEM((G, CHUNK), jnp.int32),
            pltpu.VMEM((2, CHUNK, width), jnp.float32),
            pltpu.VMEM((ZCH, width), jnp.float32),
            pltpu.VMEM_SHARED((NACC, width), jnp.float32),
            pltpu.SemaphoreType.DMA,
            pltpu.SemaphoreType.DMA,
        ],
    )
    def agg(p_hbm, src_hbm, dst_hbm, out_hbm,
            src_v, dst_v, rows_v, zbuf, acc, sem0, sem1):
        c = lax.axis_index("c")
        s = lax.axis_index("s")
        wid = c * NS + s
        _fill(zbuf, ZCH, width, 0.0)
        _zero_acc(acc, zbuf, s, width)
        plsc.subcore_barrier()

        sems = (sem0, sem1)

        def outer(g, carry):
            base = wid * CPT + g * G
            pltpu.sync_copy(src_hbm.at[pl.ds(base, G)], src_v)
            pltpu.sync_copy(dst_hbm.at[pl.ds(base, G)], dst_v)
            for half in range(G // 2):
                descs = []
                for b in range(2):
                    j = half * 2 + b
                    descs.append(pltpu.async_copy(
                        p_hbm.at[src_v.at[j]], rows_v.at[b], sems[b]))
                for b in range(2):
                    j = half * 2 + b
                    descs[b].wait()
                    pltpu.sync_copy(rows_v.at[b],
                                    acc.at[dst_v.at[j]], add=True)
            return carry

        lax.fori_loop(0, CPT // G, outer, 0)
        plsc.subcore_barrier()
        _copy_out(acc, out_hbm, c, s, width)

    return agg


def _make_cnt():
    @functools.partial(
        pl.kernel,
        out_type=jax.ShapeDtypeStruct((NC, N, C16), jnp.float32),
        mesh=_mesh(),
        compiler_params=pltpu.CompilerParams(use_tc_tiling_on_sc=False),
        scratch_types=[
            pltpu.VMEM((CPT, CHUNK), jnp.int32),
            pltpu.VMEM((CHUNK, C16), jnp.float32),
            pltpu.VMEM((ZCH, C16), jnp.float32),
            pltpu.VMEM_SHARED((NACC, C16), jnp.float32),
        ],
    )
    def cnt(dst_hbm, out_hbm, dst_v, ones_v, zbuf, acc):
        c = lax.axis_index("c")
        s = lax.axis_index("s")
        wid = c * NS + s
        _fill(zbuf, ZCH, C16, 0.0)
        _fill(ones_v, CHUNK, C16, 1.0)
        _zero_acc(acc, zbuf, s, C16)
        pltpu.sync_copy(dst_hbm.at[pl.ds(wid * CPT, CPT)], dst_v)
        plsc.subcore_barrier()

        def body(j, carry):
            pltpu.sync_copy(ones_v, acc.at[dst_v.at[j]], add=True)
            return carry

        lax.fori_loop(0, CPT, body, 0)
        plsc.subcore_barrier()
        _copy_out(acc, out_hbm, c, s, C16)

    return cnt


_agg128 = _make_agg(D)
_agg16 = _make_agg(C16)
_cnt = _make_cnt()


def _tc_in(x, Wl, Wr, b):
    def body(x_ref, wl_ref, wr_ref, b_ref, p_ref, q_ref):
        xb = x_ref[...]
        p_ref[...] = jnp.dot(xb, wl_ref[...],
                             preferred_element_type=jnp.float32)
        q_ref[...] = jnp.dot(xb, wr_ref[...],
                             preferred_element_type=jnp.float32) + b_ref[...]

    return pl.pallas_call(
        body,
        grid=(N // BLK,),
        in_specs=[
            pl.BlockSpec((BLK, D), lambda i: (i, 0)),
            pl.BlockSpec((D, D), lambda i: (0, 0)),
            pl.BlockSpec((D, D), lambda i: (0, 0)),
            pl.BlockSpec((1, D), lambda i: (0, 0)),
        ],
        out_specs=[
            pl.BlockSpec((BLK, D), lambda i: (i, 0)),
            pl.BlockSpec((BLK, D), lambda i: (i, 0)),
        ],
        out_shape=[jax.ShapeDtypeStruct((N, D), jnp.float32)] * 2,
    )(x, Wl, Wr, b.reshape(1, D))


def _tc_mid(aggp, cntp, q, Wl, Wr, b):
    wout = Wl.shape[1]

    def body(agg_ref, cnt_ref, q_ref, wl_ref, wr_ref, b_ref, p_ref, q2_ref):
        a = agg_ref[0] + agg_ref[1]
        cval = cnt_ref[0, :, 0:1] + cnt_ref[1, :, 0:1]
        rinv = 1.0 / jnp.maximum(cval, 1.0)
        h = jnp.maximum(a * rinv + q_ref[...], 0.0)
        p_ref[...] = jnp.dot(h, wl_ref[...],
                             preferred_element_type=jnp.float32)
        q2_ref[...] = jnp.dot(h, wr_ref[...],
                              preferred_element_type=jnp.float32) + b_ref[...]

    return pl.pallas_call(
        body,
        grid=(N // BLK,),
        in_specs=[
            pl.BlockSpec((NC, BLK, D), lambda i: (0, i, 0)),
            pl.BlockSpec((NC, BLK, C16), lambda i: (0, i, 0)),
            pl.BlockSpec((BLK, D), lambda i: (i, 0)),
            pl.BlockSpec((D, wout), lambda i: (0, 0)),
            pl.BlockSpec((D, wout), lambda i: (0, 0)),
            pl.BlockSpec((1, wout), lambda i: (0, 0)),
        ],
        out_specs=[
            pl.BlockSpec((BLK, wout), lambda i: (i, 0)),
            pl.BlockSpec((BLK, wout), lambda i: (i, 0)),
        ],
        out_shape=[jax.ShapeDtypeStruct((N, wout), jnp.float32)] * 2,
    )(aggp, cntp, q, Wl, Wr, b.reshape(1, wout))


def _tc_out(aggp, cntp, q):
    def body(agg_ref, cnt_ref, q_ref, o_ref):
        a = agg_ref[0] + agg_ref[1]
        cval = cnt_ref[0, :, 0:1] + cnt_ref[1, :, 0:1]
        rinv = 1.0 / jnp.maximum(cval, 1.0)
        o_ref[...] = a * rinv + q_ref[...]

    return pl.pallas_call(
        body,
        grid=(N // BLK,),
        in_specs=[
            pl.BlockSpec((NC, BLK, C16), lambda i: (0, i, 0)),
            pl.BlockSpec((NC, BLK, C16), lambda i: (0, i, 0)),
            pl.BlockSpec((BLK, C16), lambda i: (i, 0)),
        ],
        out_specs=pl.BlockSpec((BLK, C16), lambda i: (i, 0)),
        out_shape=jax.ShapeDtypeStruct((N, C16), jnp.float32),
    )(aggp, cntp, q)


def kernel(x, edge_index, Wl1, Wr1, b1, Wl2, Wr2, b2, Wl3, Wr3, b3):
    src = edge_index[0].astype(jnp.int32)
    dst = edge_index[1].astype(jnp.int32)
    e = src.shape[0]
    pad = EP - e
    src_p = jnp.concatenate(
        [src, jnp.zeros((pad,), jnp.int32)]).reshape(NW * CPT, CHUNK)
    dst_p = jnp.concatenate(
        [dst, jnp.full((pad,), N, jnp.int32)]).reshape(NW * CPT, CHUNK)
    nclass = Wl3.shape[1]
    Wl3p = jnp.pad(Wl3, ((0, 0), (0, C16 - nclass)))
    Wr3p = jnp.pad(Wr3, ((0, 0), (0, C16 - nclass)))
    b3p = jnp.pad(b3, (0, C16 - nclass))

    cntp = _cnt(dst_p)
    p1, q1 = _tc_in(x, Wl1, Wr1, b1)
    aggp1 = _agg128(p1, src_p, dst_p)
    p2, q2 = _tc_mid(aggp1, cntp, q1, Wl2, Wr2, b2)
    aggp2 = _agg128(p2, src_p, dst_p)
    p3, q3 = _tc_mid(aggp2, cntp, q2, Wl3p, Wr3p, b3p)
    aggp3 = _agg16(p3, src_p, dst_p)
    out16 = _tc_out(aggp3, cntp, q3)
    return out16[:, :nclass]


# speedup vs baseline: 5.0749x; 1.1354x over previous
import functools

import jax
import jax.numpy as jnp
from jax import lax
from jax.experimental import pallas as pl
from jax.experimental.pallas import tpu as pltpu
from jax.experimental.pallas import tpu_sc as plsc

N = 10000
D = 128
C16 = 16
NC = 2
NS = 16
NW = NC * NS
CHUNK = 128
CPT = 80
G = 8
EP = NW * CPT * CHUNK
NACC = N + 8
ZCH = 24
BLK = 1000


def _mesh():
    return plsc.VectorSubcoreMesh(
        core_axis_name="c", subcore_axis_name="s",
        num_cores=NC, num_subcores=NS)


def _fill(ref, rows, width, value):
    v = jnp.full((16,), value, jnp.float32)
    for i in range(rows):
        for j in range(width // 16):
            ref[i, pl.ds(j * 16, 16)] = v


def _zero_acc(acc, zbuf, s, width):
    for j in range(416 // NS):
        idx = s * (416 // NS) + j
        pltpu.sync_copy(zbuf, acc.at[pl.ds(idx * ZCH, ZCH)])

    @pl.when(s == NS - 1)
    def _():
        pltpu.sync_copy(zbuf, acc.at[pl.ds(416 * ZCH, ZCH)])


def _copy_out(acc, out_hbm, c, s, width):
    for j in range(416 // NS):
        idx = s * (416 // NS) + j
        pltpu.sync_copy(acc.at[pl.ds(idx * ZCH, ZCH)],
                        out_hbm.at[c, pl.ds(idx * ZCH, ZCH)])

    @pl.when(s == NS - 1)
    def _():
        pltpu.sync_copy(acc.at[pl.ds(416 * ZCH, 16)],
                        out_hbm.at[c, pl.ds(416 * ZCH, 16)])


def _make_agg(width, cpt0):
    cpt1 = 2 * CPT - cpt0
    assert cpt0 % G == 0 and cpt1 % G == 0

    @functools.partial(
        pl.kernel,
        out_type=jax.ShapeDtypeStruct((NC, N, width), jnp.float32),
        mesh=_mesh(),
        compiler_params=pltpu.CompilerParams(
            use_tc_tiling_on_sc=(width == D)),
        scratch_types=[
            pltpu.VMEM((G, CHUNK), jnp.int32),
            pltpu.VMEM((G, CHUNK), jnp.int32),
            pltpu.VMEM((2, CHUNK, width), jnp.float32),
            pltpu.VMEM((ZCH, width), jnp.float32),
            pltpu.VMEM_SHARED((NACC, width), jnp.float32),
            pltpu.SemaphoreType.DMA,
            pltpu.SemaphoreType.DMA,
        ],
    )
    def agg(p_hbm, src_hbm, dst_hbm, out_hbm,
            src_v, dst_v, rows_v, zbuf, acc, sem0, sem1):
        c = lax.axis_index("c")
        s = lax.axis_index("s")
        _fill(zbuf, ZCH, width, 0.0)
        _zero_acc(acc, zbuf, s, width)
        plsc.subcore_barrier()

        base_chunk = jnp.where(c == 0, s * cpt0, NS * cpt0 + s * cpt1)
        ngroups = jnp.where(c == 0, cpt0 // G, cpt1 // G)
        sems = (sem0, sem1)

        def outer(g, carry):
            base = base_chunk + g * G
            pltpu.sync_copy(src_hbm.at[pl.ds(base, G)], src_v)
            pltpu.sync_copy(dst_hbm.at[pl.ds(base, G)], dst_v)
            for half in range(G // 2):
                descs = []
                for b in range(2):
                    j = half * 2 + b
                    descs.append(pltpu.async_copy(
                        p_hbm.at[src_v.at[j]], rows_v.at[b], sems[b]))
                for b in range(2):
                    j = half * 2 + b
                    descs[b].wait()
                    pltpu.sync_copy(rows_v.at[b],
                                    acc.at[dst_v.at[j]], add=True)
            return carry

        lax.fori_loop(0, ngroups, outer, 0)
        plsc.subcore_barrier()
        _copy_out(acc, out_hbm, c, s, width)

    return agg


def _make_cnt():
    @functools.partial(
        pl.kernel,
        out_type=jax.ShapeDtypeStruct((NC, N, C16), jnp.float32),
        mesh=_mesh(),
        compiler_params=pltpu.CompilerParams(use_tc_tiling_on_sc=False),
        scratch_types=[
            pltpu.VMEM((CPT, CHUNK), jnp.int32),
            pltpu.VMEM((CHUNK, C16), jnp.float32),
            pltpu.VMEM((ZCH, C16), jnp.float32),
            pltpu.VMEM_SHARED((NACC, C16), jnp.float32),
        ],
    )
    def cnt(dst_hbm, out_hbm, dst_v, ones_v, zbuf, acc):
        c = lax.axis_index("c")
        s = lax.axis_index("s")
        wid = c * NS + s
        _fill(zbuf, ZCH, C16, 0.0)
        _fill(ones_v, CHUNK, C16, 1.0)
        _zero_acc(acc, zbuf, s, C16)
        pltpu.sync_copy(dst_hbm.at[pl.ds(wid * CPT, CPT)], dst_v)
        plsc.subcore_barrier()

        def body(j, carry):
            pltpu.sync_copy(ones_v, acc.at[dst_v.at[j]], add=True)
            return carry

        lax.fori_loop(0, CPT, body, 0)
        plsc.subcore_barrier()
        _copy_out(acc, out_hbm, c, s, C16)

    return cnt


_agg128 = _make_agg(D, 120)
_agg16 = _make_agg(C16, 96)
_cnt = _make_cnt()


def _tc_in(x, Wl, Wr, b):
    def body(x_ref, wl_ref, wr_ref, b_ref, p_ref, q_ref):
        xb = x_ref[...]
        p_ref[...] = jnp.dot(xb, wl_ref[...],
                             preferred_element_type=jnp.float32)
        q_ref[...] = jnp.dot(xb, wr_ref[...],
                             preferred_element_type=jnp.float32) + b_ref[...]

    return pl.pallas_call(
        body,
        grid=(N // BLK,),
        in_specs=[
            pl.BlockSpec((BLK, D), lambda i: (i, 0)),
            pl.BlockSpec((D, D), lambda i: (0, 0)),
            pl.BlockSpec((D, D), lambda i: (0, 0)),
            pl.BlockSpec((1, D), lambda i: (0, 0)),
        ],
        out_specs=[
            pl.BlockSpec((BLK, D), lambda i: (i, 0)),
            pl.BlockSpec((BLK, D), lambda i: (i, 0)),
        ],
        out_shape=[jax.ShapeDtypeStruct((N, D), jnp.float32)] * 2,
    )(x, Wl, Wr, b.reshape(1, D))


def _tc_mid(aggp, cntp, q, Wl, Wr, b):
    wout = Wl.shape[1]

    def body(agg_ref, cnt_ref, q_ref, wl_ref, wr_ref, b_ref, p_ref, q2_ref):
        a = agg_ref[0] + agg_ref[1]
        cval = cnt_ref[0, :, 0:1] + cnt_ref[1, :, 0:1]
        rinv = 1.0 / jnp.maximum(cval, 1.0)
        h = jnp.maximum(a * rinv + q_ref[...], 0.0)
        p_ref[...] = jnp.dot(h, wl_ref[...],
                             preferred_element_type=jnp.float32)
        q2_ref[...] = jnp.dot(h, wr_ref[...],
                              preferred_element_type=jnp.float32) + b_ref[...]

    return pl.pallas_call(
        body,
        grid=(N // BLK,),
        in_specs=[
            pl.BlockSpec((NC, BLK, D), lambda i: (0, i, 0)),
            pl.BlockSpec((NC, BLK, C16), lambda i: (0, i, 0)),
            pl.BlockSpec((BLK, D), lambda i: (i, 0)),
            pl.BlockSpec((D, wout), lambda i: (0, 0)),
            pl.BlockSpec((D, wout), lambda i: (0, 0)),
            pl.BlockSpec((1, wout), lambda i: (0, 0)),
        ],
        out_specs=[
            pl.BlockSpec((BLK, wout), lambda i: (i, 0)),
            pl.BlockSpec((BLK, wout), lambda i: (i, 0)),
        ],
        out_shape=[jax.ShapeDtypeStruct((N, wout), jnp.float32)] * 2,
    )(aggp, cntp, q, Wl, Wr, b.reshape(1, wout))


def _tc_out(aggp, cntp, q):
    def body(agg_ref, cnt_ref, q_ref, o_ref):
        a = agg_ref[0] + agg_ref[1]
        cval = cnt_ref[0, :, 0:1] + cnt_ref[1, :, 0:1]
        rinv = 1.0 / jnp.maximum(cval, 1.0)
        o_ref[...] = a * rinv + q_ref[...]

    return pl.pallas_call(
        body,
        grid=(N // BLK,),
        in_specs=[
            pl.BlockSpec((NC, BLK, C16), lambda i: (0, i, 0)),
            pl.BlockSpec((NC, BLK, C16), lambda i: (0, i, 0)),
            pl.BlockSpec((BLK, C16), lambda i: (i, 0)),
        ],
        out_specs=pl.BlockSpec((BLK, C16), lambda i: (i, 0)),
        out_shape=jax.ShapeDtypeStruct((N, C16), jnp.float32),
    )(aggp, cntp, q)


def kernel(x, edge_index, Wl1, Wr1, b1, Wl2, Wr2, b2, Wl3, Wr3, b3):
    src = edge_index[0].astype(jnp.int32)
    dst = edge_index[1].astype(jnp.int32)
    e = src.shape[0]
    pad = EP - e
    src_p = jnp.concatenate(
        [src, jnp.zeros((pad,), jnp.int32)]).reshape(NW * CPT, CHUNK)
    dst_p = jnp.concatenate(
        [dst, jnp.full((pad,), N, jnp.int32)]).reshape(NW * CPT, CHUNK)
    nclass = Wl3.shape[1]
    Wl3p = jnp.pad(Wl3, ((0, 0), (0, C16 - nclass)))
    Wr3p = jnp.pad(Wr3, ((0, 0), (0, C16 - nclass)))
    b3p = jnp.pad(b3, (0, C16 - nclass))

    cntp = _cnt(dst_p)
    p1, q1 = _tc_in(x, Wl1, Wr1, b1)
    aggp1 = _agg128(p1, src_p, dst_p)
    p2, q2 = _tc_mid(aggp1, cntp, q1, Wl2, Wr2, b2)
    aggp2 = _agg128(p2, src_p, dst_p)
    p3, q3 = _tc_mid(aggp2, cntp, q2, Wl3p, Wr3p, b3p)
    aggp3 = _agg16(p3, src_p, dst_p)
    out16 = _tc_out(aggp3, cntp, q3)
    return out16[:, :nclass]
